# baseline (speedup 1.0000x reference)
import functools

import jax
import jax.numpy as jnp
from jax import lax
from jax.experimental import pallas as pl
from jax.experimental.pallas import tpu as pltpu
from jax.experimental.pallas import tpu_sc as plsc

_NWORDS = 1000000
_NPAD = 1 << 20
_EMB = 64
_NTAGS = 1000
_BATCH = 4096
_HIST = 200
_TOTAL = _BATCH * _HIST

_NC = 2
_NS = 16
_NW = _NC * _NS
_PER_TILE = _TOTAL // _NW
_IDX_ROWS = _PER_TILE // 128
_ZCHUNK = 16384
_SLICE = _NPAD // _NS

_R = 8192
_NB = (_NWORDS + _R - 1) // _R


def _sc_hist_body(words_hbm, out_hbm, idx_v, ones_v, zbuf, counts_sh):
    cid = lax.axis_index("c")
    sid = lax.axis_index("s")
    wid = sid * _NC + cid

    zeros16 = jnp.zeros((16,), jnp.float32)
    ones16 = jnp.full((16,), 1.0, jnp.float32)

    def fill_z(i, _):
        zbuf[pl.ds(i * 16, 16)] = zeros16
        return 0

    lax.fori_loop(0, _ZCHUNK // 16, fill_z, 0)

    def fill_o(i, _):
        ones_v[pl.ds(i * 16, 16)] = ones16
        return 0

    lax.fori_loop(0, 8, fill_o, 0)

    def zero_counts(k, _):
        pltpu.sync_copy(
            zbuf, counts_sh.at[pl.ds(sid * _SLICE + k * _ZCHUNK, _ZCHUNK)]
        )
        return 0

    lax.fori_loop(0, _SLICE // _ZCHUNK, zero_counts, 0)
    plsc.subcore_barrier()

    pltpu.sync_copy(words_hbm.at[pl.ds(wid * _IDX_ROWS, _IDX_ROWS)], idx_v)

    def scatter(j, _):
        pltpu.sync_copy(ones_v, counts_sh.at[idx_v.at[j]], add=True)
        return 0

    lax.fori_loop(0, _IDX_ROWS, scatter, 0)
    plsc.subcore_barrier()

    pltpu.sync_copy(
        counts_sh.at[pl.ds(sid * _SLICE, _SLICE)],
        out_hbm.at[cid, pl.ds(sid * _SLICE, _SLICE)],
    )


_sc_hist = functools.partial(
    pl.kernel,
    mesh=plsc.VectorSubcoreMesh(core_axis_name="c", subcore_axis_name="s"),
    out_type=jax.ShapeDtypeStruct((_NC, _NPAD), jnp.float32),
    scratch_types=[
        pltpu.VMEM((_IDX_ROWS, 128), jnp.int32),
        pltpu.VMEM((128,), jnp.float32),
        pltpu.VMEM((_ZCHUNK,), jnp.float32),
        pltpu.VMEM_SHARED((_NPAD,), jnp.float32),
    ],
)(_sc_hist_body)


def _tc_body(c_ref, t_ref, w0_ref, b0_ref, w1_ref, b1_ref, wout_ref,
             bout_ref, o_ref, acc):
    g = pl.program_id(0)
    c = c_ref[0:1, :] + c_ref[1:2, :]
    c2 = jnp.transpose(c)

    def contrib(t):
        return jnp.sum(t * c2, axis=0, keepdims=True)

    @pl.when(g == 0)
    def _():
        acc[...] = jnp.zeros((1, _EMB), jnp.float32)

    @pl.when(g < _NB - 1)
    def _():
        acc[...] += contrib(t_ref[...])

    @pl.when(g == _NB - 1)
    def _():
        valid = _NWORDS - (_NB - 1) * _R
        rows = lax.broadcasted_iota(jnp.int32, (_R, _EMB), 0)
        t = jnp.where(rows < valid, t_ref[...], 0.0)
        acc[...] += contrib(t)

        s = jnp.sum(acc[...])
        h = jnp.tanh(s * w0_ref[...] + b0_ref[...])
        h = jnp.tanh(
            lax.dot_general(
                h, w1_ref[...], (((1,), (0,)), ((), ())),
                preferred_element_type=jnp.float32,
                precision=lax.Precision.HIGHEST,
            )
            + b1_ref[...]
        )
        o_ref[...] = (
            lax.dot_general(
                h, wout_ref[...], (((1,), (0,)), ((), ())),
                preferred_element_type=jnp.float32,
                precision=lax.Precision.HIGHEST,
            )
            + bout_ref[...]
        )


_tc_reduce_mlp = pl.pallas_call(
    _tc_body,
    grid=(_NB,),
    in_specs=[
        pl.BlockSpec((_NC, _R), lambda g: (0, g)),
        pl.BlockSpec((_R, _EMB), lambda g: (g, 0)),
        pl.BlockSpec((1, _EMB), lambda g: (0, 0)),
        pl.BlockSpec((1, _EMB), lambda g: (0, 0)),
        pl.BlockSpec((_EMB, _EMB), lambda g: (0, 0)),
        pl.BlockSpec((1, _EMB), lambda g: (0, 0)),
        pl.BlockSpec((_EMB, _NTAGS), lambda g: (0, 0)),
        pl.BlockSpec((1, _NTAGS), lambda g: (0, 0)),
    ],
    out_specs=pl.BlockSpec((1, _NTAGS), lambda g: (0, 0)),
    out_shape=jax.ShapeDtypeStruct((1, _NTAGS), jnp.float32),
    scratch_shapes=[pltpu.VMEM((1, _EMB), jnp.float32)],
)


def kernel(words, emb_table, W0, b0, W1, b1, Wout, bout):
    words2 = words.astype(jnp.int32).reshape(_TOTAL // 128, 128)
    counts = _sc_hist(words2)
    return _tc_reduce_mlp(
        counts,
        emb_table,
        W0,
        b0.reshape(1, _EMB),
        W1,
        b1.reshape(1, _EMB),
        Wout,
        bout.reshape(1, _NTAGS),
    )


# speedup vs baseline: 1.2351x; 1.2351x over previous
import functools

import jax
import jax.numpy as jnp
from jax import lax
from jax.experimental import pallas as pl
from jax.experimental.pallas import tpu as pltpu
from jax.experimental.pallas import tpu_sc as plsc

_NWORDS = 1000000
_NPAD = 1 << 20
_EMB = 64
_NTAGS = 1000
_BATCH = 4096
_HIST = 200
_TOTAL = _BATCH * _HIST

_NC = 2
_NS = 16
_NW = _NC * _NS
_PER_TILE = _TOTAL // _NW
_IDX_ROWS = _PER_TILE // 128
_ZCHUNK = 16384
_SLICE = _NPAD // _NS

_R = 16384
_NB = (_NWORDS + _R - 1) // _R


def _sc_hist_body(words_hbm, out_hbm, idx_v, ones_v, zbuf, counts_sh):
    cid = lax.axis_index("c")
    sid = lax.axis_index("s")
    wid = sid * _NC + cid

    zeros16 = jnp.zeros((16,), jnp.float32)
    ones16 = jnp.full((16,), 1.0, jnp.float32)

    def fill_z(i, _):
        zbuf[pl.ds(i * 16, 16)] = zeros16
        return 0

    lax.fori_loop(0, _ZCHUNK // 16, fill_z, 0)

    def fill_o(i, _):
        ones_v[pl.ds(i * 16, 16)] = ones16
        return 0

    lax.fori_loop(0, 8, fill_o, 0)

    def zero_counts(k, _):
        pltpu.sync_copy(
            zbuf, counts_sh.at[pl.ds(sid * _SLICE + k * _ZCHUNK, _ZCHUNK)]
        )
        return 0

    lax.fori_loop(0, _SLICE // _ZCHUNK, zero_counts, 0)
    plsc.subcore_barrier()

    pltpu.sync_copy(words_hbm.at[pl.ds(wid * _IDX_ROWS, _IDX_ROWS)], idx_v)

    def scatter(j, _):
        pltpu.sync_copy(ones_v, counts_sh.at[idx_v.at[j]], add=True)
        return 0

    lax.fori_loop(0, _IDX_ROWS, scatter, 0)
    plsc.subcore_barrier()

    pltpu.sync_copy(
        counts_sh.at[pl.ds(sid * _SLICE, _SLICE)],
        out_hbm.at[cid, pl.ds(sid * _SLICE, _SLICE)],
    )


_sc_hist = functools.partial(
    pl.kernel,
    mesh=plsc.VectorSubcoreMesh(core_axis_name="c", subcore_axis_name="s"),
    out_type=jax.ShapeDtypeStruct((_NC, _NPAD), jnp.float32),
    scratch_types=[
        pltpu.VMEM((_IDX_ROWS, 128), jnp.int32),
        pltpu.VMEM((128,), jnp.float32),
        pltpu.VMEM((_ZCHUNK,), jnp.float32),
        pltpu.VMEM_SHARED((_NPAD,), jnp.float32),
    ],
)(_sc_hist_body)


def _tc_body(c_ref, t_ref, w0_ref, b0_ref, w1_ref, b1_ref, wout_ref,
             bout_ref, o_ref, acc):
    g = pl.program_id(0)
    c = c_ref[0:1, :] + c_ref[1:2, :]
    c_bf = c.astype(jnp.bfloat16)

    def _dot(a, b):
        return lax.dot_general(
            a, b, (((1,), (0,)), ((), ())),
            preferred_element_type=jnp.float32,
        )

    def contrib(t):
        t_hi = t.astype(jnp.bfloat16)
        t_lo = (t - t_hi.astype(jnp.float32)).astype(jnp.bfloat16)
        return _dot(c_bf, t_hi) + _dot(c_bf, t_lo)

    @pl.when(g == 0)
    def _():
        acc[...] = jnp.zeros((1, _EMB), jnp.float32)

    @pl.when(g < _NB - 1)
    def _():
        acc[...] += contrib(t_ref[...])

    @pl.when(g == _NB - 1)
    def _():
        valid = _NWORDS - (_NB - 1) * _R
        rows = lax.broadcasted_iota(jnp.int32, (_R, _EMB), 0)
        t = jnp.where(rows < valid, t_ref[...], 0.0)
        acc[...] += contrib(t)

        s = jnp.sum(acc[...])
        h = jnp.tanh(s * w0_ref[...] + b0_ref[...])
        h = jnp.tanh(
            lax.dot_general(
                h, w1_ref[...], (((1,), (0,)), ((), ())),
                preferred_element_type=jnp.float32,
                precision=lax.Precision.HIGHEST,
            )
            + b1_ref[...]
        )
        o_ref[...] = (
            lax.dot_general(
                h, wout_ref[...], (((1,), (0,)), ((), ())),
                preferred_element_type=jnp.float32,
                precision=lax.Precision.HIGHEST,
            )
            + bout_ref[...]
        )


_tc_reduce_mlp = pl.pallas_call(
    _tc_body,
    grid=(_NB,),
    in_specs=[
        pl.BlockSpec((_NC, _R), lambda g: (0, g)),
        pl.BlockSpec((_R, _EMB), lambda g: (g, 0)),
        pl.BlockSpec((1, _EMB), lambda g: (0, 0)),
        pl.BlockSpec((1, _EMB), lambda g: (0, 0)),
        pl.BlockSpec((_EMB, _EMB), lambda g: (0, 0)),
        pl.BlockSpec((1, _EMB), lambda g: (0, 0)),
        pl.BlockSpec((_EMB, _NTAGS), lambda g: (0, 0)),
        pl.BlockSpec((1, _NTAGS), lambda g: (0, 0)),
    ],
    out_specs=pl.BlockSpec((1, _NTAGS), lambda g: (0, 0)),
    out_shape=jax.ShapeDtypeStruct((1, _NTAGS), jnp.float32),
    scratch_shapes=[pltpu.VMEM((1, _EMB), jnp.float32)],
)


def kernel(words, emb_table, W0, b0, W1, b1, Wout, bout):
    words2 = words.astype(jnp.int32).reshape(_TOTAL // 128, 128)
    counts = _sc_hist(words2)
    return _tc_reduce_mlp(
        counts,
        emb_table,
        W0,
        b0.reshape(1, _EMB),
        W1,
        b1.reshape(1, _EMB),
        Wout,
        bout.reshape(1, _NTAGS),
    )
